# Initial kernel scaffold; baseline (speedup 1.0000x reference)
import jax
import jax.numpy as jnp
from jax.experimental import pallas as pl


def kernel(x, pe):
    raise NotImplementedError("write your pallas kernel here")


# speedup vs baseline: 6.1471x; 6.1471x over previous
import jax
import jax.numpy as jnp
from jax.experimental import pallas as pl


_BATCH_BLOCK = 128


def _add_pe_kernel(x_ref, pe_ref, o_ref):
    o_ref[...] = x_ref[...] + pe_ref[...]


def kernel(x, pe):
    bsz, seq_len, d_model = x.shape
    row = seq_len * d_model
    x2 = x.reshape(bsz, row)
    pe2 = pe.reshape(1, row)

    grid = bsz // _BATCH_BLOCK
    out = pl.pallas_call(
        _add_pe_kernel,
        grid=(grid,),
        in_specs=[
            pl.BlockSpec((_BATCH_BLOCK, row), lambda i: (i, 0)),
            pl.BlockSpec((1, row), lambda i: (0, 0)),
        ],
        out_specs=pl.BlockSpec((_BATCH_BLOCK, row), lambda i: (i, 0)),
        out_shape=jax.ShapeDtypeStruct((bsz, row), x.dtype),
    )(x2, pe2)
    return out.reshape(bsz, seq_len, d_model)


# speedup vs baseline: 6.1677x; 1.0033x over previous
import jax
import jax.numpy as jnp
from jax.experimental import pallas as pl


_BATCH_BLOCK = 256


def _add_pe_kernel(x_ref, pe_ref, o_ref):
    o_ref[...] = x_ref[...] + pe_ref[...]


def kernel(x, pe):
    bsz, seq_len, d_model = x.shape
    row = seq_len * d_model
    x2 = x.reshape(bsz, row)
    pe2 = pe.reshape(1, row)

    grid = bsz // _BATCH_BLOCK
    out = pl.pallas_call(
        _add_pe_kernel,
        grid=(grid,),
        in_specs=[
            pl.BlockSpec((_BATCH_BLOCK, row), lambda i: (i, 0)),
            pl.BlockSpec((1, row), lambda i: (0, 0)),
        ],
        out_specs=pl.BlockSpec((_BATCH_BLOCK, row), lambda i: (i, 0)),
        out_shape=jax.ShapeDtypeStruct((bsz, row), x.dtype),
    )(x2, pe2)
    return out.reshape(bsz, seq_len, d_model)
